# baseline (speedup 1.0000x reference)
import dataclasses
import functools

import jax
import jax.numpy as jnp
from jax import lax
from jax.experimental import pallas as pl
from jax.experimental.pallas import tpu as pltpu
from jax.experimental.pallas import tpu_sc as plsc

_L = 16
_NW = 32
_D = 128
_WIN = 128


def _sc_kernel(x_hbm, mask_hbm, w_hbm, b_hbm, out_hbm,
               mask_v, idx_v, rows_v, o_vmem, w_v, b_v, sem):
    chunk = mask_v.shape[0]
    wid = lax.axis_index("s") * 2 + lax.axis_index("c")
    base = wid * chunk

    pltpu.sync_copy(mask_hbm.at[pl.ds(base, chunk)], mask_v)
    pltpu.sync_copy(w_hbm, w_v)
    pltpu.sync_copy(b_hbm, b_v)

    lane = lax.iota(jnp.int32, _L)
    zero_f = jnp.zeros((_L,), jnp.float32)
    zero_i = jnp.zeros((_L,), jnp.int32)

    @pl.loop(0, chunk // _L)
    def _(i):
        o_vmem[pl.ds(i * _L, _L)] = zero_f
        idx_v[pl.ds(i * _L, _L)] = zero_i

    def compact_body(i, count_vec):
        mvec = mask_v[pl.ds(i * _L, _L)]
        mbool = mvec != 0
        pos = plsc.cumsum(mvec) + count_vec - 1
        gidx = lane + (base + i * _L)
        plsc.store_scatter(idx_v, [pos], gidx, mask=mbool)
        return count_vec + plsc.all_reduce_population_count(mbool)

    count_vec = lax.fori_loop(0, chunk // _L, compact_body,
                              jnp.zeros((_L,), jnp.int32))
    count_s = jnp.max(count_vec)
    nwin = (count_s + _WIN - 1) // _WIN

    wregs = [w_v[pl.ds(k * _L, _L)] for k in range(_D // _L)]
    b_vec = b_v[...]

    @pl.loop(0, nwin)
    def _(j):
        pltpu.async_copy(x_hbm.at[idx_v.at[pl.ds(j * _WIN, _WIN)]],
                         rows_v, sem).wait()

        @pl.loop(0, _WIN // _L)
        def _(g):
            y_acc = zero_f
            for r in range(_L):
                row = g * _L + r
                acc = zero_f
                for k in range(_D // _L):
                    acc = acc + rows_v[row, pl.ds(k * _L, _L)] * wregs[k]
                tot = jnp.full((_L,), jnp.sum(acc), jnp.float32)
                y_acc = jnp.where(lane == r, tot, y_acc)
            pos_g = j * _WIN + g * _L
            widx = idx_v[pl.ds(pos_g, _L)]
            valid = (lane + pos_g) < count_vec
            plsc.store_scatter(o_vmem, [widx - base], y_acc + b_vec,
                               mask=valid)

    pltpu.sync_copy(o_vmem, out_hbm.at[pl.ds(base, chunk)])


def _sc_call(x2d, mask_i32, w_vec, b_vec16):
    R = x2d.shape[0]
    chunk = R // _NW
    mesh = plsc.VectorSubcoreMesh(core_axis_name="c", subcore_axis_name="s")
    cp = pltpu.CompilerParams()
    if "needs_layout_passes" in pltpu.CompilerParams.__dataclass_fields__:
        cp = dataclasses.replace(cp, needs_layout_passes=False)
    kern = functools.partial(
        pl.kernel,
        compiler_params=cp,
        out_type=jax.ShapeDtypeStruct((R,), jnp.float32),
        mesh=mesh,
        scratch_types=[
            pltpu.VMEM((chunk,), jnp.int32),
            pltpu.VMEM((chunk,), jnp.int32),
            pltpu.VMEM((_WIN, _D), jnp.float32),
            pltpu.VMEM((chunk,), jnp.float32),
            pltpu.VMEM((_D,), jnp.float32),
            pltpu.VMEM((_L,), jnp.float32),
            pltpu.SemaphoreType.DMA,
        ],
    )(_sc_kernel)
    return kern(x2d, mask_i32, w_vec, b_vec16)


def kernel(embs, masks, W, b):
    B, N, D = embs.shape
    R = B * N
    x2d = embs.reshape(R, D)
    mask_i32 = masks.reshape(R).astype(jnp.int32)
    w_vec = W.reshape(D).astype(jnp.float32)
    b_vec16 = jnp.full((_L,), b[0], jnp.float32)
    out = _sc_call(x2d, mask_i32, w_vec, b_vec16)
    return out.reshape(B, N)
